# baseline (speedup 1.0000x reference)
import jax
import jax.numpy as jnp
from jax.experimental import pallas as pl
from jax.experimental.pallas import tpu as pltpu

_K = 64
_R = 8
_N = 32768
_W = 512
_NCH = _N // _W
_MIN_I32 = -2147483648


def _orderable(x):
    b = pltpu.bitcast(x, jnp.int32)
    return b ^ (jax.lax.shift_right_arithmetic(b, 31) & 0x7FFFFFFF)


def _topk_kernel(x_ref, res_ref, idx_ref):
    key = _orderable(x_ref[...])

    def count_ge(c):
        return jnp.sum((key >= c).astype(jnp.int32), axis=1, keepdims=True)

    prefix = jnp.where(count_ge(jnp.zeros((_R, 1), jnp.int32)) >= _K,
                       jnp.zeros((_R, 1), jnp.int32),
                       jnp.full((_R, 1), _MIN_I32, jnp.int32))

    def bit_body(i, prefix):
        cand = prefix | jax.lax.shift_left(1, 30 - i)
        return jnp.where(count_ge(cand) >= _K, cand, prefix)

    t = jax.lax.fori_loop(0, 31, bit_body, prefix)
    n_gt = jnp.sum((key > t).astype(jnp.int32), axis=1, keepdims=True)
    budget = (_K - n_gt).astype(jnp.float32)

    tri = (jax.lax.broadcasted_iota(jnp.int32, (_W, _W), 0)
           < jax.lax.broadcasted_iota(jnp.int32, (_W, _W), 1)).astype(jnp.bfloat16)
    lane_w = jax.lax.broadcasted_iota(jnp.int32, (_R, _W), 1)
    p_iota3 = jax.lax.broadcasted_iota(jnp.int32, (_R, _W, _K), 2)

    def chunk_body(c, carry):
        acc, gt_carry, eq_carry = carry
        off = c * _W
        xc = x_ref[:, pl.ds(off, _W)]
        kc = _orderable(xc)
        gt = kc > t
        eq = kc == t
        planes2 = jnp.concatenate(
            [gt.astype(jnp.bfloat16), eq.astype(jnp.bfloat16)], axis=0)
        pref = jax.lax.dot_general(planes2, tri, (((1,), (0,)), ((), ())),
                                   preferred_element_type=jnp.float32)
        g_gt = pref[:_R] + gt_carry
        g_eq = pref[_R:] + eq_carry
        gt_carry = g_gt[:, _W - 1:] + gt.astype(jnp.float32)[:, _W - 1:]
        eq_carry = g_eq[:, _W - 1:] + eq.astype(jnp.float32)[:, _W - 1:]
        sel = gt | (eq & (g_eq < budget))
        res_ref[:, pl.ds(off, _W)] = jnp.where(sel, jnp.maximum(xc, 0.0), 0.0)
        pos = (g_gt + jnp.minimum(g_eq, budget)).astype(jnp.int32)
        pos = jnp.where(sel, pos, -1)
        oh = (pos[:, :, None] == p_iota3).astype(jnp.bfloat16)
        gidx = off + lane_w
        planes = jnp.stack(
            [(kc & 255).astype(jnp.bfloat16),
             (jax.lax.shift_right_logical(kc, 8) & 255).astype(jnp.bfloat16),
             (jax.lax.shift_right_logical(kc, 16) & 255).astype(jnp.bfloat16),
             (jax.lax.shift_right_logical(kc, 24) & 255).astype(jnp.bfloat16),
             jax.lax.shift_right_logical(gidx, 8).astype(jnp.bfloat16),
             (gidx & 255).astype(jnp.bfloat16)],
            axis=1)
        got = jax.lax.dot_general(planes, oh, (((2,), (1,)), ((0,), (0,))),
                                  preferred_element_type=jnp.float32)
        return acc + got, gt_carry, eq_carry

    acc0 = jnp.zeros((_R, 6, _K), jnp.float32)
    z1 = jnp.zeros((_R, 1), jnp.float32)
    acc, _, _ = jax.lax.fori_loop(0, _NCH, chunk_body, (acc0, z1, z1))

    accs = acc.astype(jnp.int32)
    ck = ((jax.lax.shift_left(accs[:, 3, :], 24))
          | (jax.lax.shift_left(accs[:, 2, :], 16))
          | (jax.lax.shift_left(accs[:, 1, :], 8))
          | accs[:, 0, :])
    cidx = jax.lax.shift_left(accs[:, 4, :], 8) | accs[:, 5, :]
    m_i = jax.lax.broadcasted_iota(jnp.int32, (_R, _K, _K), 2)
    j_i = jax.lax.broadcasted_iota(jnp.int32, (_R, _K, _K), 1)
    km = ck[:, None, :]
    kj = ck[:, :, None]
    ahead = (km > kj) | ((km == kj) & (m_i < j_i))
    rank = jnp.sum(ahead.astype(jnp.int32), axis=2)
    p_i = jax.lax.broadcasted_iota(jnp.int32, (_R, _K, _K), 1)
    hit = rank[:, None, :] == p_i
    idx_ref[...] = jnp.sum(jnp.where(hit, cidx[:, None, :], 0), axis=2)


def kernel(x):
    rows, n = x.shape
    result, idx = pl.pallas_call(
        _topk_kernel,
        grid=(rows // _R,),
        in_specs=[pl.BlockSpec((_R, n), lambda i: (i, 0))],
        out_specs=[
            pl.BlockSpec((_R, n), lambda i: (i, 0)),
            pl.BlockSpec((_R, _K), lambda i: (i, 0)),
        ],
        out_shape=[
            jax.ShapeDtypeStruct((rows, n), x.dtype),
            jax.ShapeDtypeStruct((rows, _K), jnp.int32),
        ],
    )(x)
    return (result, idx)


# speedup vs baseline: 1.5492x; 1.5492x over previous
import jax
import jax.numpy as jnp
from jax.experimental import pallas as pl
from jax.experimental.pallas import tpu as pltpu

_K = 64
_R = 8
_N = 32768
_W = 512
_NCH = _N // _W
_MIN_I32 = -2147483648


def _orderable(x):
    b = pltpu.bitcast(x, jnp.int32)
    return b ^ (jax.lax.shift_right_arithmetic(b, 31) & 0x7FFFFFFF)


def _topk_kernel(x_ref, res_ref, idx_ref):
    key = _orderable(x_ref[...])

    def count_ge(c):
        return jnp.sum(jnp.where(key >= c, 1, 0), axis=1, keepdims=True)

    prefix = jnp.where(count_ge(jnp.zeros((_R, 1), jnp.int32)) >= _K,
                       jnp.zeros((_R, 1), jnp.int32),
                       jnp.full((_R, 1), _MIN_I32, jnp.int32))

    def bit_body(i, prefix):
        cand = prefix | jax.lax.shift_left(1, 30 - i)
        return jnp.where(count_ge(cand) >= _K, cand, prefix)

    t = jax.lax.fori_loop(0, 31, bit_body, prefix)
    n_gt = jnp.sum(jnp.where(key > t, 1, 0), axis=1, keepdims=True)
    budget = (_K - n_gt).astype(jnp.float32)

    tri = (jax.lax.broadcasted_iota(jnp.int32, (_W, _W), 0)
           < jax.lax.broadcasted_iota(jnp.int32, (_W, _W), 1)).astype(jnp.bfloat16)
    lane_w = jax.lax.broadcasted_iota(jnp.int32, (_R, _W), 1)
    p_iota = (jax.lax.broadcasted_iota(jnp.int32, (_R, _K, _W), 1)
              .astype(jnp.bfloat16))

    def chunk_body(c, carry):
        acc, gt_carry, eq_carry = carry
        off = c * _W
        xc = x_ref[:, pl.ds(off, _W)]
        kc = _orderable(xc)
        gt = kc > t
        eq = kc == t
        planes2 = jnp.concatenate(
            [jnp.where(gt, 1.0, 0.0).astype(jnp.bfloat16),
             jnp.where(eq, 1.0, 0.0).astype(jnp.bfloat16)], axis=0)
        pref = jax.lax.dot_general(planes2, tri, (((1,), (0,)), ((), ())),
                                   preferred_element_type=jnp.float32)
        g_gt = pref[:_R] + gt_carry
        g_eq = pref[_R:] + eq_carry
        gt_carry = g_gt[:, _W - 1:] + jnp.where(gt[:, _W - 1:], 1.0, 0.0)
        eq_carry = g_eq[:, _W - 1:] + jnp.where(eq[:, _W - 1:], 1.0, 0.0)
        sel = gt | (eq & (g_eq < budget))
        res_ref[:, pl.ds(off, _W)] = jnp.where(sel, jnp.maximum(xc, 0.0), 0.0)
        pos = g_gt + jnp.minimum(g_eq, budget)
        pos_bf = jnp.where(sel, pos, -1.0).astype(jnp.bfloat16)
        oh = jnp.where(pos_bf[:, None, :] == p_iota,
                       jnp.bfloat16(1), jnp.bfloat16(0))
        gidx = off + lane_w
        planes = jnp.stack(
            [(kc & 255).astype(jnp.bfloat16),
             (jax.lax.shift_right_logical(kc, 8) & 255).astype(jnp.bfloat16),
             (jax.lax.shift_right_logical(kc, 16) & 255).astype(jnp.bfloat16),
             (jax.lax.shift_right_logical(kc, 24) & 255).astype(jnp.bfloat16),
             jax.lax.shift_right_logical(gidx, 8).astype(jnp.bfloat16),
             (gidx & 255).astype(jnp.bfloat16)],
            axis=1)
        got = jax.lax.dot_general(planes, oh, (((2,), (2,)), ((0,), (0,))),
                                  preferred_element_type=jnp.float32)
        return acc + got, gt_carry, eq_carry

    acc0 = jnp.zeros((_R, 6, _K), jnp.float32)
    z1 = jnp.zeros((_R, 1), jnp.float32)
    acc, _, _ = jax.lax.fori_loop(0, _NCH, chunk_body, (acc0, z1, z1))

    accs = acc.astype(jnp.int32)
    ck = ((jax.lax.shift_left(accs[:, 3, :], 24))
          | (jax.lax.shift_left(accs[:, 2, :], 16))
          | (jax.lax.shift_left(accs[:, 1, :], 8))
          | accs[:, 0, :])
    cidx = jax.lax.shift_left(accs[:, 4, :], 8) | accs[:, 5, :]
    m_i = jax.lax.broadcasted_iota(jnp.int32, (_R, _K, _K), 2)
    j_i = jax.lax.broadcasted_iota(jnp.int32, (_R, _K, _K), 1)
    km = ck[:, None, :]
    kj = ck[:, :, None]
    ahead = (km > kj) | ((km == kj) & (m_i < j_i))
    rank = jnp.sum(jnp.where(ahead, 1, 0), axis=2)
    p_i = jax.lax.broadcasted_iota(jnp.int32, (_R, _K, _K), 1)
    hit = rank[:, None, :] == p_i
    idx_ref[...] = jnp.sum(jnp.where(hit, cidx[:, None, :], 0), axis=2)


def kernel(x):
    rows, n = x.shape
    result, idx = pl.pallas_call(
        _topk_kernel,
        grid=(rows // _R,),
        in_specs=[pl.BlockSpec((_R, n), lambda i: (i, 0))],
        out_specs=[
            pl.BlockSpec((_R, n), lambda i: (i, 0)),
            pl.BlockSpec((_R, _K), lambda i: (i, 0)),
        ],
        out_shape=[
            jax.ShapeDtypeStruct((rows, n), x.dtype),
            jax.ShapeDtypeStruct((rows, _K), jnp.int32),
        ],
    )(x)
    return (result, idx)
